# baseline (speedup 1.0000x reference)
import functools

import jax
import jax.numpy as jnp
from jax import lax
from jax.experimental import pallas as pl
from jax.experimental.pallas import tpu as pltpu
from jax.experimental.pallas import tpu_sc as plsc

N = 10000
B = 16384
E = 320000
D_DRUG = 128
D_PROT = 256
H = 128
W = 144
CH = 256
NTILES = 16
NACC = 10240
ZROWS = NACC // NTILES
NI = 80
ECHP = NI * NTILES
EPAD = ECHP * CH
GCH = B // (NTILES * CH)
ZBLK = 128


_PROJ_BLK = 2000


def _proj_body(dx, px, wd, bd, wp, bp, out):
    f32 = jnp.float32
    hd = jnp.dot(dx[...], wd[...], preferred_element_type=f32) + bd[...]
    hp = jnp.dot(px[...], wp[...], preferred_element_type=f32) + bp[...]
    ones = jnp.ones((hd.shape[0], W - H), f32)
    out[0] = jnp.concatenate([hd, ones], axis=1)
    out[1] = jnp.concatenate([hp, ones], axis=1)


def _project(drug_x, protein_x, W_dlin, b_dlin, W_plin, b_plin):
    return pl.pallas_call(
        _proj_body,
        grid=(N // _PROJ_BLK,),
        in_specs=[
            pl.BlockSpec((_PROJ_BLK, D_DRUG), lambda i: (i, 0)),
            pl.BlockSpec((_PROJ_BLK, D_PROT), lambda i: (i, 0)),
            pl.BlockSpec((D_DRUG, H), lambda i: (0, 0)),
            pl.BlockSpec((1, H), lambda i: (0, 0)),
            pl.BlockSpec((D_PROT, H), lambda i: (0, 0)),
            pl.BlockSpec((1, H), lambda i: (0, 0)),
        ],
        out_specs=pl.BlockSpec((2, _PROJ_BLK, W), lambda i: (0, i, 0)),
        out_shape=jax.ShapeDtypeStruct((2, N, W), jnp.float32),
    )(drug_x, protein_x, W_dlin, b_dlin.reshape(1, H), W_plin,
      b_plin.reshape(1, H))


def _sc_body(table, esd, gtid, gaid, zrows,
             accg, tabg, acc_sh, idx_v, rows_v, sem_r):
    c = lax.axis_index("c")
    s = lax.axis_index("s")
    ebase = s * NI

    pltpu.sync_copy(zrows, rows_v.at[pl.ds(0, ZBLK)])
    zbase = s * ZROWS
    for k in range(ZROWS // ZBLK):
        pltpu.sync_copy(rows_v.at[pl.ds(0, ZBLK)],
                        acc_sh.at[pl.ds(zbase + k * ZBLK, ZBLK)])
    plsc.subcore_barrier()

    def edge_chunk(i, carry):
        pltpu.sync_copy(esd.at[c, ebase + i], idx_v)
        pltpu.async_copy(table.at[idx_v.at[0]], rows_v, sem_r).wait()
        pltpu.sync_copy(rows_v, acc_sh.at[idx_v.at[1]], add=True)
        return carry

    lax.fori_loop(0, NI, edge_chunk, 0)

    gbase = s * GCH

    def sample_phase(src, idx_hbm, out_ref):
        for k in range(GCH):
            pltpu.sync_copy(idx_hbm.at[c, gbase + k], idx_v.at[0])
            pltpu.async_copy(src.at[idx_v.at[0]], rows_v, sem_r).wait()
            pltpu.sync_copy(rows_v,
                            out_ref.at[c, pl.ds((gbase + k) * CH, CH)])

    sample_phase(table, gtid, tabg)
    plsc.subcore_barrier()
    sample_phase(acc_sh, gaid, accg)


def _segment_gather(table, esd, gtid, gaid, zrows):
    call = pl.kernel(
        _sc_body,
        mesh=plsc.VectorSubcoreMesh(core_axis_name="c", subcore_axis_name="s"),
        compiler_params=pltpu.CompilerParams(use_tc_tiling_on_sc=False),
        out_type=[
            jax.ShapeDtypeStruct((2, B, W), jnp.float32),
            jax.ShapeDtypeStruct((2, B, W), jnp.float32),
        ],
        scratch_types=[
            pltpu.VMEM_SHARED((NACC, W), jnp.float32),
            pltpu.VMEM((2, CH), jnp.int32),
            pltpu.VMEM((CH, W), jnp.float32),
            pltpu.SemaphoreType.DMA,
        ],
    )
    return call(table, esd, gtid, gaid, zrows)


_FIN_BLK = 2048


def _final_body(accg, tabg, wldp, bldp, wrdp, wlpd, blpd, wrpd,
                wfc1, bfc1, wfc2, bfc2, out):
    f32 = jnp.float32
    p_acc = accg[0]
    d_acc = accg[1]
    hd = tabg[0, :, :H]
    hp = tabg[1, :, :H]
    mean_p = p_acc[:, :H] / jnp.maximum(p_acc[:, H:H + 1], 1.0)
    mean_d = d_acc[:, :H] / jnp.maximum(d_acc[:, H:H + 1], 1.0)
    d = (jnp.dot(mean_d, wlpd[...], preferred_element_type=f32) + blpd[...]
         + jnp.dot(hd, wrpd[...], preferred_element_type=f32))
    p = (jnp.dot(mean_p, wldp[...], preferred_element_type=f32) + bldp[...]
         + jnp.dot(hp, wrdp[...], preferred_element_type=f32))
    w1 = wfc1[...]
    h = jnp.maximum(
        jnp.dot(d, w1[:H], preferred_element_type=f32)
        + jnp.dot(p, w1[H:], preferred_element_type=f32) + bfc1[...], 0.0)
    out[...] = jnp.dot(h, wfc2[...], preferred_element_type=f32) + bfc2[...]


def _finalize(accg, tabg, Wl_dp, bl_dp, Wr_dp, Wl_pd, bl_pd, Wr_pd,
              W_fc1, b_fc1, W_fc2, b_fc2):
    full = lambda i: (0, 0)
    return pl.pallas_call(
        _final_body,
        grid=(B // _FIN_BLK,),
        in_specs=[
            pl.BlockSpec((2, _FIN_BLK, W), lambda i: (0, i, 0)),
            pl.BlockSpec((2, _FIN_BLK, W), lambda i: (0, i, 0)),
            pl.BlockSpec((H, H), full),
            pl.BlockSpec((1, H), full),
            pl.BlockSpec((H, H), full),
            pl.BlockSpec((H, H), full),
            pl.BlockSpec((1, H), full),
            pl.BlockSpec((H, H), full),
            pl.BlockSpec((2 * H, H), full),
            pl.BlockSpec((1, H), full),
            pl.BlockSpec((H, 1), full),
            pl.BlockSpec((1, 1), full),
        ],
        out_specs=pl.BlockSpec((_FIN_BLK, 1), lambda i: (i, 0)),
        out_shape=jax.ShapeDtypeStruct((B, 1), jnp.float32),
    )(accg, tabg, Wl_dp, bl_dp.reshape(1, H), Wr_dp, Wl_pd,
      bl_pd.reshape(1, H), Wr_pd, W_fc1, b_fc1.reshape(1, H), W_fc2,
      b_fc2.reshape(1, 1))


def kernel(drug_x, protein_x, edge_index_dp, edge_index_pd, drug_idx,
           protein_idx, W_dlin, b_dlin, W_plin, b_plin, Wl_dp, bl_dp, Wr_dp,
           Wl_pd, bl_pd, Wr_pd, W_fc1, b_fc1, W_fc2, b_fc2):
    ei_dp = edge_index_dp.astype(jnp.int32)
    ei_pd = edge_index_pd.astype(jnp.int32)
    didx = drug_idx.astype(jnp.int32)
    pidx = protein_idx.astype(jnp.int32)

    npad = EPAD - E
    pad_src = jnp.zeros((npad,), jnp.int32)
    pad_dst = N + (jnp.arange(npad, dtype=jnp.int32) % (NACC - N))
    src2 = jnp.stack([
        jnp.concatenate([ei_dp[0], pad_src]),
        jnp.concatenate([ei_pd[0] + N, pad_src]),
    ]).reshape(2, ECHP, 1, CH)
    dst2 = jnp.stack([
        jnp.concatenate([ei_dp[1], pad_dst]),
        jnp.concatenate([ei_pd[1], pad_dst]),
    ]).reshape(2, ECHP, 1, CH)
    esd = jnp.concatenate([src2, dst2], axis=2)
    gtid = jnp.stack([didx, pidx + N]).reshape(2, B // CH, CH)
    gaid = jnp.stack([pidx, didx]).reshape(2, B // CH, CH)
    zrows = jnp.zeros((ZBLK, W), jnp.float32)

    table = _project(drug_x, protein_x, W_dlin, b_dlin,
                     W_plin, b_plin).reshape(2 * N, W)
    accg, tabg = _segment_gather(table, esd, gtid, gaid, zrows)
    out = _finalize(accg, tabg, Wl_dp, bl_dp, Wr_dp, Wl_pd, bl_pd, Wr_pd,
                    W_fc1, b_fc1, W_fc2, b_fc2)
    return out.reshape(B)


# speedup vs baseline: 1.0893x; 1.0893x over previous
import functools

import jax
import jax.numpy as jnp
from jax import lax
from jax.experimental import pallas as pl
from jax.experimental.pallas import tpu as pltpu
from jax.experimental.pallas import tpu_sc as plsc

N = 10000
B = 16384
E = 320000
D_DRUG = 128
D_PROT = 256
H = 128
CW = 8
CH = 256
NTILES = 16
NACC = 10240
ZROWS = NACC // NTILES
NI = 80
ECHP = NI * NTILES
EPAD = ECHP * CH
GCH = B // (NTILES * CH)
ZBLK = 128


_PROJ_BLK = 2000


def _proj_body(dx, px, wd, bd, wp, bp, out):
    f32 = jnp.float32
    out[0] = jnp.dot(dx[...], wd[...], preferred_element_type=f32) + bd[...]
    out[1] = jnp.dot(px[...], wp[...], preferred_element_type=f32) + bp[...]


def _project(drug_x, protein_x, W_dlin, b_dlin, W_plin, b_plin):
    return pl.pallas_call(
        _proj_body,
        grid=(N // _PROJ_BLK,),
        in_specs=[
            pl.BlockSpec((_PROJ_BLK, D_DRUG), lambda i: (i, 0)),
            pl.BlockSpec((_PROJ_BLK, D_PROT), lambda i: (i, 0)),
            pl.BlockSpec((D_DRUG, H), lambda i: (0, 0)),
            pl.BlockSpec((1, H), lambda i: (0, 0)),
            pl.BlockSpec((D_PROT, H), lambda i: (0, 0)),
            pl.BlockSpec((1, H), lambda i: (0, 0)),
        ],
        out_specs=pl.BlockSpec((2, _PROJ_BLK, H), lambda i: (0, i, 0)),
        out_shape=jax.ShapeDtypeStruct((2, N, H), jnp.float32),
    )(drug_x, protein_x, W_dlin, b_dlin.reshape(1, H), W_plin,
      b_plin.reshape(1, H))


def _sc_body(table, esd, gtid, gaid, zrows, zcnt, ones_hbm,
             accg, tabg, cntg, acc_sh, cnt_sh, idx_v, rows_v, cnt_v, sem_r):
    c = lax.axis_index("c")
    s = lax.axis_index("s")
    ebase = s * NI

    pltpu.sync_copy(zrows, rows_v.at[pl.ds(0, ZBLK)])
    pltpu.sync_copy(zcnt, cnt_v.at[pl.ds(0, ZBLK)])
    zbase = s * ZROWS
    for k in range(ZROWS // ZBLK):
        pltpu.sync_copy(rows_v.at[pl.ds(0, ZBLK)],
                        acc_sh.at[pl.ds(zbase + k * ZBLK, ZBLK)])
        pltpu.sync_copy(cnt_v.at[pl.ds(0, ZBLK)],
                        cnt_sh.at[pl.ds(zbase + k * ZBLK, ZBLK)])
    pltpu.sync_copy(ones_hbm, cnt_v)
    plsc.subcore_barrier()

    def edge_chunk(i, carry):
        pltpu.sync_copy(esd.at[c, ebase + i], idx_v)
        pltpu.async_copy(table.at[idx_v.at[0]], rows_v, sem_r).wait()
        pltpu.sync_copy(rows_v, acc_sh.at[idx_v.at[1]], add=True)
        pltpu.sync_copy(cnt_v, cnt_sh.at[idx_v.at[1]], add=True)
        return carry

    lax.fori_loop(0, NI, edge_chunk, 0)

    gbase = s * GCH

    def sample_phase(src, idx_hbm, out_ref, buf):
        for k in range(GCH):
            pltpu.sync_copy(idx_hbm.at[c, gbase + k], idx_v.at[0])
            pltpu.async_copy(src.at[idx_v.at[0]], buf, sem_r).wait()
            pltpu.sync_copy(buf, out_ref.at[c, pl.ds((gbase + k) * CH, CH)])

    sample_phase(table, gtid, tabg, rows_v)
    plsc.subcore_barrier()
    sample_phase(acc_sh, gaid, accg, rows_v)
    sample_phase(cnt_sh, gaid, cntg, cnt_v)


def _segment_gather(table, esd, gtid, gaid, zrows, zcnt, ones_hbm):
    call = pl.kernel(
        _sc_body,
        mesh=plsc.VectorSubcoreMesh(core_axis_name="c", subcore_axis_name="s"),
        compiler_params=pltpu.CompilerParams(use_tc_tiling_on_sc=False),
        out_type=[
            jax.ShapeDtypeStruct((2, B, H), jnp.float32),
            jax.ShapeDtypeStruct((2, B, H), jnp.float32),
            jax.ShapeDtypeStruct((2, B, CW), jnp.float32),
        ],
        scratch_types=[
            pltpu.VMEM_SHARED((NACC, H), jnp.float32),
            pltpu.VMEM_SHARED((NACC, CW), jnp.float32),
            pltpu.VMEM((2, CH), jnp.int32),
            pltpu.VMEM((CH, H), jnp.float32),
            pltpu.VMEM((CH, CW), jnp.float32),
            pltpu.SemaphoreType.DMA,
        ],
    )
    return call(table, esd, gtid, gaid, zrows, zcnt, ones_hbm)


_FIN_BLK = 2048


def _final_body(accg, tabg, cntg, wldp, bldp, wrdp, wlpd, blpd, wrpd,
                wfc1, bfc1, wfc2, bfc2, out):
    f32 = jnp.float32
    hd = tabg[0]
    hp = tabg[1]
    mean_p = accg[0] / jnp.maximum(cntg[0, :, :1], 1.0)
    mean_d = accg[1] / jnp.maximum(cntg[1, :, :1], 1.0)
    d = (jnp.dot(mean_d, wlpd[...], preferred_element_type=f32) + blpd[...]
         + jnp.dot(hd, wrpd[...], preferred_element_type=f32))
    p = (jnp.dot(mean_p, wldp[...], preferred_element_type=f32) + bldp[...]
         + jnp.dot(hp, wrdp[...], preferred_element_type=f32))
    w1 = wfc1[...]
    h = jnp.maximum(
        jnp.dot(d, w1[:H], preferred_element_type=f32)
        + jnp.dot(p, w1[H:], preferred_element_type=f32) + bfc1[...], 0.0)
    out[...] = jnp.dot(h, wfc2[...], preferred_element_type=f32) + bfc2[...]


def _finalize(accg, tabg, cntg, Wl_dp, bl_dp, Wr_dp, Wl_pd, bl_pd, Wr_pd,
              W_fc1, b_fc1, W_fc2, b_fc2):
    full = lambda i: (0, 0)
    return pl.pallas_call(
        _final_body,
        grid=(B // _FIN_BLK,),
        in_specs=[
            pl.BlockSpec((2, _FIN_BLK, H), lambda i: (0, i, 0)),
            pl.BlockSpec((2, _FIN_BLK, H), lambda i: (0, i, 0)),
            pl.BlockSpec((2, _FIN_BLK, CW), lambda i: (0, i, 0)),
            pl.BlockSpec((H, H), full),
            pl.BlockSpec((1, H), full),
            pl.BlockSpec((H, H), full),
            pl.BlockSpec((H, H), full),
            pl.BlockSpec((1, H), full),
            pl.BlockSpec((H, H), full),
            pl.BlockSpec((2 * H, H), full),
            pl.BlockSpec((1, H), full),
            pl.BlockSpec((H, 1), full),
            pl.BlockSpec((1, 1), full),
        ],
        out_specs=pl.BlockSpec((_FIN_BLK, 1), lambda i: (i, 0)),
        out_shape=jax.ShapeDtypeStruct((B, 1), jnp.float32),
    )(accg, tabg, cntg, Wl_dp, bl_dp.reshape(1, H), Wr_dp, Wl_pd,
      bl_pd.reshape(1, H), Wr_pd, W_fc1, b_fc1.reshape(1, H), W_fc2,
      b_fc2.reshape(1, 1))


def kernel(drug_x, protein_x, edge_index_dp, edge_index_pd, drug_idx,
           protein_idx, W_dlin, b_dlin, W_plin, b_plin, Wl_dp, bl_dp, Wr_dp,
           Wl_pd, bl_pd, Wr_pd, W_fc1, b_fc1, W_fc2, b_fc2):
    ei_dp = edge_index_dp.astype(jnp.int32)
    ei_pd = edge_index_pd.astype(jnp.int32)
    didx = drug_idx.astype(jnp.int32)
    pidx = protein_idx.astype(jnp.int32)

    npad = EPAD - E
    pad_src = jnp.zeros((npad,), jnp.int32)
    pad_dst = N + (jnp.arange(npad, dtype=jnp.int32) % (NACC - N))
    src2 = jnp.stack([
        jnp.concatenate([ei_dp[0], pad_src]),
        jnp.concatenate([ei_pd[0] + N, pad_src]),
    ]).reshape(2, ECHP, 1, CH)
    dst2 = jnp.stack([
        jnp.concatenate([ei_dp[1], pad_dst]),
        jnp.concatenate([ei_pd[1], pad_dst]),
    ]).reshape(2, ECHP, 1, CH)
    esd = jnp.concatenate([src2, dst2], axis=2)
    gtid = jnp.stack([didx, pidx + N]).reshape(2, B // CH, CH)
    gaid = jnp.stack([pidx, didx]).reshape(2, B // CH, CH)
    zrows = jnp.zeros((ZBLK, H), jnp.float32)
    zcnt = jnp.zeros((ZBLK, CW), jnp.float32)
    ones_hbm = jnp.ones((CH, CW), jnp.float32)

    table = _project(drug_x, protein_x, W_dlin, b_dlin,
                     W_plin, b_plin).reshape(2 * N, H)
    accg, tabg, cntg = _segment_gather(table, esd, gtid, gaid,
                                       zrows, zcnt, ones_hbm)
    out = _finalize(accg, tabg, cntg, Wl_dp, bl_dp, Wr_dp, Wl_pd, bl_pd,
                    Wr_pd, W_fc1, b_fc1, W_fc2, b_fc2)
    return out.reshape(B)
